# Initial kernel scaffold; baseline (speedup 1.0000x reference)
import jax
import jax.numpy as jnp
from jax.experimental import pallas as pl


def kernel(x1, edge_index1, batch1, x2, edge_index2, batch2, cell, params):
    raise NotImplementedError("write your pallas kernel here")


# speedup vs baseline: 1.0000x; 1.0000x over previous
import jax
import jax.numpy as jnp
from jax.experimental import pallas as pl
from jax.experimental.pallas import tpu as pltpu

NUM_GRAPHS = 128


def _lrelu(x, s=0.01):
    return jnp.where(x >= 0, x, s * x)


def _gat_conv(x, src, dst, W, a_s, a_d, b):
    N = x.shape[0]
    h = x @ W
    e = _lrelu((h @ a_s)[src] + (h @ a_d)[dst], 0.2)
    m = jax.ops.segment_max(e, dst, num_segments=N)
    m = jnp.where(jnp.isfinite(m), m, 0.0)
    ex = jnp.exp(e - m[dst])
    den = jax.ops.segment_sum(ex, dst, num_segments=N)
    alpha = ex / (den[dst] + 1e-16)
    return jax.ops.segment_sum(h[src] * alpha[:, None], dst, num_segments=N) + b


def _drug_fem(x, edge_index, batch, p):
    N = x.shape[0]
    loop = jnp.arange(N, dtype=edge_index.dtype)
    src = jnp.concatenate([edge_index[0], loop])
    dst = jnp.concatenate([edge_index[1], loop])
    for i in range(3):
        x = _lrelu(_gat_conv(x, src, dst, p[f"W{i}"], p[f"as{i}"], p[f"ad{i}"], p[f"b{i}"]))
    s = jax.nn.softmax(x @ p["att_w"] + p["att_b"], axis=0)
    g = jax.ops.segment_sum(s * x, batch, num_segments=NUM_GRAPHS)
    h = _lrelu((g @ p["fc1_w"] + p["fc1_b"]) * p["bn1_g"] + p["bn1_b"])
    return h @ p["fc2_w"] + p["fc2_b"]


def _tail_kernel(h1_ref, h2_ref, cell_ref, cp_ref_list, mp_ref_list, sp_ref_list, out_ref):
    cw1, cb1, cg1, cbe1, cw2, cb2, cg2, cbe2, cw3, cb3 = cp_ref_list
    h1 = h1_ref[...]
    h2 = h2_ref[...]
    cell = cell_ref[...]
    v = cell / (jnp.sqrt(jnp.sum(cell * cell, axis=1, keepdims=True)) + 1e-12)
    c = _lrelu((v @ cw1[...] + cb1[...]) * cg1[...] + cbe1[...])
    c = _lrelu((c @ cw2[...] + cb2[...]) * cg2[...] + cbe2[...])
    c = c @ cw3[...] + cb3[...]
    x = jnp.concatenate([h1, h2, c], axis=1)
    for i in range(2):
        gw, gb, nw, nb, lw, lb = mp_ref_list[6 * i:6 * i + 6]
        gate = jax.nn.sigmoid(x @ gw[...] + gb[...])
        nl = _lrelu(x @ nw[...] + nb[...])
        lin = x @ lw[...] + lb[...]
        x = gate * nl + (1.0 - gate) * lin
    w1, b1, w2, b2, w3, b3 = sp_ref_list
    h = _lrelu(x @ w1[...] + b1[...])
    h = _lrelu(h @ w2[...] + b2[...])
    out_ref[...] = h @ w3[...] + b3[...]


def _tail(h1, h2, cell, params):
    cp = params["cell"]
    mp = params["mfic"]
    sp = params["syn"]
    cp_list = [cp["w1"], cp["b1"], cp["g1"], cp["be1"], cp["w2"], cp["b2"],
               cp["g2"], cp["be2"], cp["w3"], cp["b3"]]
    mp_list = []
    for i in range(2):
        mp_list += [mp[f"gw{i}"], mp[f"gb{i}"], mp[f"nw{i}"], mp[f"nb{i}"],
                    mp[f"lw{i}"], mp[f"lb{i}"]]
    sp_list = [sp["w1"], sp["b1"], sp["w2"], sp["b2"], sp["w3"], sp["b3"]]
    return pl.pallas_call(
        _tail_kernel,
        out_shape=jax.ShapeDtypeStruct((NUM_GRAPHS, 2), jnp.float32),
    )(h1, h2, cell, cp_list, mp_list, sp_list)


def kernel(x1, edge_index1, batch1, x2, edge_index2, batch2, cell, params):
    h1 = _drug_fem(x1, edge_index1, batch1, params["d1"])
    h2 = _drug_fem(x2, edge_index2, batch2, params["d2"])
    return _tail(h1, h2, cell, params)


# speedup vs baseline: 18.2298x; 18.2290x over previous
import functools

import jax
import jax.numpy as jnp
from jax import lax
from jax.experimental import pallas as pl
from jax.experimental.pallas import tpu as pltpu
from jax.experimental.pallas import tpu_sc as plsc

NG = 128
N = 10000
NPAD = 10240
E_REAL = 650000
NTILES = 16
EPT = 40960
E_PAD = EPT * NTILES
SL = NPAD // NTILES
CH1 = 2048
CH3 = 256
FW = 64

_MESH = plsc.VectorSubcoreMesh(
    core_axis_name="c", subcore_axis_name="s", num_cores=2, num_subcores=16)
_SC_PARAMS = pltpu.CompilerParams(
    use_tc_tiling_on_sc=False, needs_layout_passes=False)


def _lrelu(x, s=0.01):
    return jnp.where(x >= 0, x, s * x)


def _conv_body(src_hbm, dst2_hbm, hs_hbm, hd_hbm, mx_hbm, h_hbm,
               out_hbm,
               hs_v, hd_v, mx_v, den_v, dent_v, src1_v, dst1_v,
               src_v, dst2_v, al_v, rows_v, zb_v,
               sh_den, sh_out, sem):
    c = lax.axis_index("c")
    s = lax.axis_index("s")
    iota16 = lax.iota(jnp.int32, 16)
    zeros16 = jnp.zeros((16,), jnp.float32)
    zeros16i = jnp.zeros((16,), jnp.int32)

    pltpu.sync_copy(hs_hbm.at[pl.ds(c * NPAD, NPAD)], hs_v)
    pltpu.sync_copy(hd_hbm.at[pl.ds(c * NPAD, NPAD)], hd_v)
    pltpu.sync_copy(mx_hbm.at[pl.ds(c * NPAD, 16)], mx_v)
    mx = mx_v[...]

    def zb0(i, carry):
        den_v[pl.ds(i * 16, 16)] = zeros16
        return carry
    lax.fori_loop(0, NPAD // 16, zb0, 0)

    ebase = (c * NTILES + s) * EPT

    def p1(ci, carry):
        off = ebase + ci * CH1
        pltpu.sync_copy(src_hbm.at[pl.ds(off, CH1)], src1_v)
        pltpu.sync_copy(dst2_hbm.at[pl.ds(off // 128, CH1 // 128)], dst1_v)

        def eb(g, carry2):
            isrc = src1_v[pl.ds(g * 16, 16)]
            idst = plsc.load_gather(
                dst1_v, [jnp.full((16,), g // 8, jnp.int32),
                         (g % 8) * 16 + iota16])
            a = plsc.load_gather(hs_v, [isrc])
            b = plsc.load_gather(hd_v, [idst])
            z = a + b
            e = jnp.where(z >= 0.0, z, 0.2 * z)
            m = jnp.maximum(b + mx, 0.0)
            plsc.addupdate_scatter(den_v, [idst], jnp.exp(e - m))
            return carry2
        lax.fori_loop(0, CH1 // 16, eb, 0)
        return carry
    lax.fori_loop(0, EPT // CH1, p1, 0)

    pltpu.sync_copy(den_v, sh_den.at[s])

    def zq(i, carry):
        plsc.store_scatter(zb_v, [zeros16i + i // (FW // 16),
                                  (i % (FW // 16)) * 16 + iota16], zeros16)
        return carry
    lax.fori_loop(0, 16 * FW // 16, zq, 0)
    for k in range(SL // 16):
        pltpu.sync_copy(zb_v, sh_out.at[pl.ds(s * SL + k * 16, 16)])
    plsc.subcore_barrier()

    def zb1(i, carry):
        den_v[pl.ds(i * 16, 16)] = zeros16
        return carry
    lax.fori_loop(0, NPAD // 16, zb1, 0)
    for t in range(NTILES):
        pltpu.sync_copy(sh_den.at[t], dent_v)

        def ad(i, carry):
            sl = pl.ds(i * 16, 16)
            den_v[sl] = den_v[sl] + dent_v[sl]
            return carry
        lax.fori_loop(0, NPAD // 16, ad, 0)

    coff = c * NPAD

    def p3(ci, carry):
        off = ebase + ci * CH3
        pltpu.sync_copy(src_hbm.at[pl.ds(off, CH3)], src_v)
        pltpu.sync_copy(dst2_hbm.at[pl.ds(off // 128, CH3 // 128)], dst2_v)

        def ab(g, carry2):
            isrc = src_v[pl.ds(g * 16, 16)]
            idst = plsc.load_gather(
                dst2_v, [jnp.full((16,), g // 8, jnp.int32),
                         (g % 8) * 16 + iota16])
            a = plsc.load_gather(hs_v, [isrc])
            b = plsc.load_gather(hd_v, [idst])
            z = a + b
            e = jnp.where(z >= 0.0, z, 0.2 * z)
            m = jnp.maximum(b + mx, 0.0)
            exv = jnp.exp(e - m)
            dv = plsc.load_gather(den_v, [idst])
            al_v[pl.ds(g * 16, 16)] = exv / (dv + 1e-16)
            src_v[pl.ds(g * 16, 16)] = isrc + coff
            return carry2
        lax.fori_loop(0, CH3 // 16, ab, 0)
        pltpu.async_copy(h_hbm.at[src_v], rows_v, sem).wait()

        def sc(e, carry2):
            er = zeros16i + e
            bc = plsc.load_gather(al_v, [er])
            for j in range(FW // 16):
                cols = j * 16 + iota16
                v = plsc.load_gather(rows_v, [er, cols])
                plsc.store_scatter(rows_v, [er, cols], v * bc)
            return carry2
        lax.fori_loop(0, CH3, sc, 0)

        for j in range(CH3 // 128):
            pltpu.sync_copy(rows_v.at[pl.ds(j * 128, 128)],
                            sh_out.at[dst2_v.at[j]], add=True)
        return carry
    lax.fori_loop(0, EPT // CH3, p3, 0)

    plsc.subcore_barrier()
    pltpu.sync_copy(sh_out.at[pl.ds(s * SL, SL)],
                    out_hbm.at[pl.ds(c * NPAD + s * SL, SL)])


_conv = pl.kernel(
    _conv_body,
    out_type=jax.ShapeDtypeStruct((2 * NPAD, FW), jnp.float32),
    mesh=_MESH,
    compiler_params=_SC_PARAMS,
    scratch_types=(
        pltpu.VMEM((NPAD,), jnp.float32),
        pltpu.VMEM((NPAD,), jnp.float32),
        pltpu.VMEM((16,), jnp.float32),
        pltpu.VMEM((NPAD,), jnp.float32),
        pltpu.VMEM((NPAD,), jnp.float32),
        pltpu.VMEM((CH1,), jnp.int32),
        pltpu.VMEM((CH1 // 128, 128), jnp.int32),
        pltpu.VMEM((CH3,), jnp.int32),
        pltpu.VMEM((CH3 // 128, 128), jnp.int32),
        pltpu.VMEM((CH3,), jnp.float32),
        pltpu.VMEM((CH3, FW), jnp.float32),
        pltpu.VMEM((16, FW), jnp.float32),
        pltpu.VMEM_SHARED((NTILES, NPAD), jnp.float32),
        pltpu.VMEM_SHARED((NPAD, FW), jnp.float32),
        pltpu.SemaphoreType.DMA,
    ),
)


def _mm_body(pre, split, p_ref, b_ref, w_ref, a_ref,
             hlo_ref, hhi_ref, hsd_ref):
    if pre:
        x = p_ref[0]
    else:
        x = _lrelu(p_ref[...] + b_ref[0])
        rid = lax.broadcasted_iota(jnp.int32, (NPAD, 1), 0)
        x = jnp.where(rid < N, x, 0.0)
    h = x @ w_ref[0]
    hsd = h @ a_ref[0]
    mxv = jnp.max(hsd[:, 0])
    ci = lax.broadcasted_iota(jnp.int32, (NPAD, 4), 1)
    hsd = jnp.where(ci == 2, mxv, hsd)
    if split:
        hlo_ref[...] = h[:, :FW]
        hhi_ref[...] = h[:, FW:]
    else:
        hlo_ref[...] = h
        hhi_ref[...] = jnp.zeros((NPAD, FW), jnp.float32)
    hsd_ref[...] = hsd


def _k_mm(pre, p, b, W, A):
    Fi, Fo = W.shape[1], W.shape[2]
    split = Fo == 2 * FW
    p_spec = (pl.BlockSpec((1, NPAD, Fi), lambda i: (i, 0, 0)) if pre
              else pl.BlockSpec((NPAD, Fi), lambda i: (i, 0)))
    return pl.pallas_call(
        functools.partial(_mm_body, pre, split),
        grid=(2,),
        in_specs=[p_spec,
                  pl.BlockSpec((1, 1, Fi), lambda i: (i, 0, 0)),
                  pl.BlockSpec((1, Fi, Fo), lambda i: (i, 0, 0)),
                  pl.BlockSpec((1, Fo, 4), lambda i: (i, 0, 0))],
        out_specs=[pl.BlockSpec((NPAD, FW), lambda i: (i, 0)),
                   pl.BlockSpec((NPAD, FW), lambda i: (i, 0)),
                   pl.BlockSpec((NPAD, 4), lambda i: (i, 0))],
        out_shape=[jax.ShapeDtypeStruct((2 * NPAD, FW), jnp.float32),
                   jax.ShapeDtypeStruct((2 * NPAD, FW), jnp.float32),
                   jax.ShapeDtypeStruct((2 * NPAD, 4), jnp.float32)],
    )(p, b, W, A)


def _post_body(plo_ref, phi_ref, b_ref, aw_ref, ab_ref, bc_ref,
               f1_ref, fb1_ref, g1_ref, be1_ref, f2_ref, fb2_ref, hf_ref):
    p = jnp.concatenate([plo_ref[...], phi_ref[...]], axis=1)
    x = _lrelu(p + b_ref[0])
    rid = lax.broadcasted_iota(jnp.int32, (NPAD, 1), 0)
    valid = rid < N
    x = jnp.where(valid, x, 0.0)
    l = x @ aw_ref[0] + ab_ref[0]
    lm = jnp.max(jnp.where(valid, l, -1e30))
    u = jnp.where(valid, jnp.exp(l - lm), 0.0)
    den = jnp.sum(u)
    onehot = (bc_ref[0] == lax.broadcasted_iota(jnp.int32, (1, NG), 1))
    w = jnp.where(onehot, u, 0.0)
    g = lax.dot_general(w, x, (((0,), (0,)), ((), ()))) / den
    hh = _lrelu((g @ f1_ref[0] + fb1_ref[0]) * g1_ref[0] + be1_ref[0])
    hf_ref[0] = hh @ f2_ref[0] + fb2_ref[0]


def _k_post(plo, phi, b, aw, ab, bc, f1, fb1, g1, be1, f2, fb2):
    args = (b, aw, ab, bc, f1, fb1, g1, be1, f2, fb2)
    specs = [pl.BlockSpec((NPAD, FW), lambda i: (i, 0)),
             pl.BlockSpec((NPAD, FW), lambda i: (i, 0))]
    specs += [pl.BlockSpec((1,) + x.shape[1:],
                           lambda i, n=x.ndim: (i,) + (0,) * (n - 1))
              for x in args]
    return pl.pallas_call(
        _post_body,
        grid=(2,),
        in_specs=specs,
        out_specs=pl.BlockSpec((1, NG, 128), lambda i: (i, 0, 0)),
        out_shape=jax.ShapeDtypeStruct((2, NG, 128), jnp.float32),
    )(plo, phi, *args)


def _tail_body(hf_ref, cell_ref, cp, mp, sp, out_ref):
    cw1, cb1, cg1, cbe1, cw2, cb2, cg2, cbe2, cw3, cb3 = cp
    h1 = hf_ref[0]
    h2 = hf_ref[1]
    cell = cell_ref[...]
    v = cell / (jnp.sqrt(jnp.sum(cell * cell, axis=1, keepdims=True)) + 1e-12)
    c = _lrelu((v @ cw1[...] + cb1[...]) * cg1[...] + cbe1[...])
    c = _lrelu((c @ cw2[...] + cb2[...]) * cg2[...] + cbe2[...])
    c = c @ cw3[...] + cb3[...]
    x = jnp.concatenate([h1, h2, c], axis=1)
    for i in range(2):
        gw, gb, nw, nb, lw, lb = mp[6 * i:6 * i + 6]
        gate = jax.nn.sigmoid(x @ gw[...] + gb[...])
        nl = _lrelu(x @ nw[...] + nb[...])
        lin = x @ lw[...] + lb[...]
        x = gate * nl + (1.0 - gate) * lin
    w1, b1, w2, b2, w3, b3 = sp
    h = _lrelu(x @ w1[...] + b1[...])
    h = _lrelu(h @ w2[...] + b2[...])
    out_ref[...] = h @ w3[...] + b3[...]


def _k_tail(hf, cell, params):
    cp = params["cell"]
    mp = params["mfic"]
    sp = params["syn"]
    cp_list = [cp["w1"], cp["b1"], cp["g1"], cp["be1"], cp["w2"], cp["b2"],
               cp["g2"], cp["be2"], cp["w3"], cp["b3"]]
    mp_list = []
    for i in range(2):
        mp_list += [mp[f"gw{i}"], mp[f"gb{i}"], mp[f"nw{i}"], mp[f"nb{i}"],
                    mp[f"lw{i}"], mp[f"lb{i}"]]
    sp_list = [sp["w1"], sp["b1"], sp["w2"], sp["b2"], sp["w3"], sp["b3"]]
    return pl.pallas_call(
        _tail_body,
        out_shape=jax.ShapeDtypeStruct((NG, 2), jnp.float32),
    )(hf, cell, cp_list, mp_list, sp_list)


def _edge_arrays(ei):
    loop = jnp.arange(N, dtype=jnp.int32)
    pad = jnp.full((E_PAD - E_REAL,), NPAD - 1, jnp.int32)
    src = jnp.concatenate([ei[0], loop, pad])
    dst = jnp.concatenate([ei[1], loop, pad])
    return src, dst


def kernel(x1, edge_index1, batch1, x2, edge_index2, batch2, cell, params):
    d1, d2 = params["d1"], params["d2"]
    src1, dst1 = _edge_arrays(edge_index1)
    src2, dst2 = _edge_arrays(edge_index2)
    src = jnp.concatenate([src1, src2])
    dst2d = jnp.concatenate([dst1, dst2]).reshape(-1, 128)
    xp = jnp.zeros((2, NPAD, 78), jnp.float32)
    xp = xp.at[:, :N].set(jnp.stack([x1, x2]))
    bc = jnp.stack([batch1, batch2]).astype(jnp.int32)
    bc = jnp.pad(bc, ((0, 0), (0, NPAD - N)))[:, :, None]

    def st(name):
        return jnp.stack([d1[name], d2[name]])

    Ws = [st(f"W{i}") for i in range(3)]
    Ws[0] = jnp.pad(Ws[0], ((0, 0), (0, 0), (0, 32)))
    Ws[1] = jnp.pad(Ws[1], ((0, 0), (0, 32), (0, 0)))
    As = []
    for i in range(3):
        pads = jnp.zeros_like(d1[f"as{i}"])
        As.append(jnp.stack(
            [jnp.stack([d1[f"as{i}"], d1[f"ad{i}"], pads, pads], axis=1),
             jnp.stack([d2[f"as{i}"], d2[f"ad{i}"], pads, pads], axis=1)]))
    As[0] = jnp.pad(As[0], ((0, 0), (0, 32), (0, 0)))
    bs = [st(f"b{i}") for i in range(3)]
    bs[0] = jnp.pad(bs[0], ((0, 0), (0, 32)))

    hlo, hhi, hsd = _k_mm(True, xp, jnp.zeros((2, 1, 78), jnp.float32),
                          Ws[0], As[0])
    out0 = _conv(src, dst2d, hsd[:, 0], hsd[:, 1], hsd[:, 2], hlo)
    hlo, hhi, hsd = _k_mm(False, out0, bs[0][:, None], Ws[1], As[1])
    out1 = _conv(src, dst2d, hsd[:, 0], hsd[:, 1], hsd[:, 2], hlo)
    hlo, hhi, hsd = _k_mm(False, out1, bs[1][:, None], Ws[2], As[2])
    out_lo = _conv(src, dst2d, hsd[:, 0], hsd[:, 1], hsd[:, 2], hlo)
    out_hi = _conv(src, dst2d, hsd[:, 0], hsd[:, 1], hsd[:, 2], hhi)

    hf = _k_post(out_lo, out_hi, bs[2][:, None], st("att_w"),
                 st("att_b")[:, None], bc,
                 st("fc1_w"), st("fc1_b")[:, None], st("bn1_g")[:, None],
                 st("bn1_b")[:, None], st("fc2_w"), st("fc2_b")[:, None])

    return _k_tail(hf, cell, params)


# speedup vs baseline: 23.4721x; 1.2876x over previous
import functools

import jax
import jax.numpy as jnp
from jax import lax
from jax.experimental import pallas as pl
from jax.experimental.pallas import tpu as pltpu
from jax.experimental.pallas import tpu_sc as plsc

NG = 128
N = 10000
NPAD = 10240
E_REAL = 650000
NTILES = 16
EPT = 40960
E_PAD = EPT * NTILES
SL = NPAD // NTILES
CH1 = 2048
CH3 = 256
FW = 64

_MESH = plsc.VectorSubcoreMesh(
    core_axis_name="c", subcore_axis_name="s", num_cores=2, num_subcores=16)
_SC_PARAMS = pltpu.CompilerParams(
    use_tc_tiling_on_sc=False, needs_layout_passes=False)


def _lrelu(x, s=0.01):
    return jnp.where(x >= 0, x, s * x)


def _conv_body(src_hbm, dst2_hbm, hs_hbm, hd_hbm, mx_hbm, h_hbm,
               out_hbm,
               hs_v, hd_v, mx_v, den_v, dent_v, src1_v, dst1_v,
               src_a, dst2_a, al_a, rows_a,
               src_b, dst2_b, al_b, rows_b, zb_v,
               sh_den, sh_out, sga, sgb, ssa, ssb):
    c = lax.axis_index("c")
    s = lax.axis_index("s")
    iota16 = lax.iota(jnp.int32, 16)
    zeros16 = jnp.zeros((16,), jnp.float32)
    zeros16i = jnp.zeros((16,), jnp.int32)

    pltpu.sync_copy(hs_hbm.at[pl.ds(c * NPAD, NPAD)], hs_v)
    pltpu.sync_copy(hd_hbm.at[pl.ds(c * NPAD, NPAD)], hd_v)
    pltpu.sync_copy(mx_hbm.at[pl.ds(c * NPAD, 16)], mx_v)
    mx = mx_v[...]

    def zb0(i, carry):
        den_v[pl.ds(i * 16, 16)] = zeros16
        return carry
    lax.fori_loop(0, NPAD // 16, zb0, 0)

    ebase = (c * NTILES + s) * EPT

    def p1(ci, carry):
        off = ebase + ci * CH1
        pltpu.sync_copy(src_hbm.at[pl.ds(off, CH1)], src1_v)
        pltpu.sync_copy(dst2_hbm.at[pl.ds(off // 128, CH1 // 128)], dst1_v)

        def eb(g, carry2):
            isrc = src1_v[pl.ds(g * 16, 16)]
            idst = plsc.load_gather(
                dst1_v, [jnp.full((16,), g // 8, jnp.int32),
                         (g % 8) * 16 + iota16])
            a = plsc.load_gather(hs_v, [isrc])
            b = plsc.load_gather(hd_v, [idst])
            z = a + b
            e = jnp.where(z >= 0.0, z, 0.2 * z)
            m = jnp.maximum(b + mx, 0.0)
            plsc.addupdate_scatter(den_v, [idst], jnp.exp(e - m))
            return carry2
        lax.fori_loop(0, CH1 // 16, eb, 0)
        return carry
    lax.fori_loop(0, EPT // CH1, p1, 0)

    pltpu.sync_copy(den_v, sh_den.at[s])

    def zq(i, carry):
        plsc.store_scatter(zb_v, [zeros16i + i // (FW // 16),
                                  (i % (FW // 16)) * 16 + iota16], zeros16)
        return carry
    lax.fori_loop(0, 16 * FW // 16, zq, 0)
    for k in range(SL // 16):
        pltpu.sync_copy(zb_v, sh_out.at[pl.ds(s * SL + k * 16, 16)])
    plsc.subcore_barrier()

    def zb1(i, carry):
        den_v[pl.ds(i * 16, 16)] = zeros16
        return carry
    lax.fori_loop(0, NPAD // 16, zb1, 0)
    for t in range(NTILES):
        for q in range(NPAD // 2048):
            pltpu.sync_copy(sh_den.at[t, pl.ds(q * 2048, 2048)], dent_v)

            def ad(i, carry):
                sl = pl.ds(q * 2048 + i * 16, 16)
                sl2 = pl.ds(i * 16, 16)
                den_v[sl] = den_v[sl] + dent_v[sl2]
                return carry
            lax.fori_loop(0, 2048 // 16, ad, 0)

    coff = c * NPAD
    NCH = EPT // CH3

    def prep(i, srcb, dst2b, alb):
        off = ebase + i * CH3
        pltpu.sync_copy(src_hbm.at[pl.ds(off, CH3)], srcb)
        pltpu.sync_copy(dst2_hbm.at[pl.ds(off // 128, CH3 // 128)], dst2b)

        def ab(g, carry2):
            isrc = srcb[pl.ds(g * 16, 16)]
            idst = plsc.load_gather(
                dst2b, [jnp.full((16,), g // 8, jnp.int32),
                        (g % 8) * 16 + iota16])
            a = plsc.load_gather(hs_v, [isrc])
            b = plsc.load_gather(hd_v, [idst])
            z = a + b
            e = jnp.where(z >= 0.0, z, 0.2 * z)
            m = jnp.maximum(b + mx, 0.0)
            exv = jnp.exp(e - m)
            dv = plsc.load_gather(den_v, [idst])
            alb[pl.ds(g * 16, 16)] = exv / (dv + 1e-16)
            srcb[pl.ds(g * 16, 16)] = isrc + coff
            return carry2
        lax.fori_loop(0, CH3 // 16, ab, 0)

    def scale(rowsb, alb):
        def scg(g, carry2):
            alv = alb[pl.ds(g * 16, 16)]
            for k in range(16):
                er = zeros16i + (g * 16 + k)
                bc = plsc.load_gather(alb, [zeros16i + g * 16 + k])
                for j in range(FW // 16):
                    cols = j * 16 + iota16
                    v = plsc.load_gather(rowsb, [er, cols])
                    plsc.store_scatter(rowsb, [er, cols], v * bc)
            return carry2
        lax.fori_loop(0, CH3 // 16, scg, 0)

    def fire_gather(srcb, rowsb, semx):
        pltpu.async_copy(h_hbm.at[srcb], rowsb, semx)

    def drain_gather(rowsb, semx):
        pltpu.make_async_copy(h_hbm.at[pl.ds(0, CH3)], rowsb, semx).wait()

    def fire_scatters(rowsb, dst2b, semx):
        for j in range(CH3 // 128):
            pltpu.async_copy(rowsb.at[pl.ds(j * 128, 128)],
                             sh_out.at[dst2b.at[j]], semx, add=True)

    def drain_scatters(rowsb, semx):
        for j in range(CH3 // 128):
            pltpu.make_async_copy(h_hbm.at[pl.ds(0, 128)],
                                  rowsb.at[pl.ds(j * 128, 128)], semx).wait()

    prep(0, src_a, dst2_a, al_a)
    fire_gather(src_a, rows_a, sga)
    prep(1, src_b, dst2_b, al_b)
    fire_gather(src_b, rows_b, sgb)

    def body(cj, carry):
        i0 = 2 * cj
        drain_gather(rows_a, sga)
        scale(rows_a, al_a)
        fire_scatters(rows_a, dst2_a, ssa)
        drain_gather(rows_b, sgb)
        scale(rows_b, al_b)
        fire_scatters(rows_b, dst2_b, ssb)
        drain_scatters(rows_a, ssa)
        prep(jnp.minimum(i0 + 2, NCH - 1), src_a, dst2_a, al_a)
        fire_gather(src_a, rows_a, sga)
        drain_scatters(rows_b, ssb)
        prep(jnp.minimum(i0 + 3, NCH - 1), src_b, dst2_b, al_b)
        fire_gather(src_b, rows_b, sgb)
        return carry
    lax.fori_loop(0, NCH // 2, body, 0)
    drain_gather(rows_a, sga)
    drain_gather(rows_b, sgb)

    plsc.subcore_barrier()
    pltpu.sync_copy(sh_out.at[pl.ds(s * SL, SL)],
                    out_hbm.at[pl.ds(c * NPAD + s * SL, SL)])


_conv = pl.kernel(
    _conv_body,
    out_type=jax.ShapeDtypeStruct((2 * NPAD, FW), jnp.float32),
    mesh=_MESH,
    compiler_params=_SC_PARAMS,
    scratch_types=(
        pltpu.VMEM((NPAD,), jnp.float32),
        pltpu.VMEM((NPAD,), jnp.float32),
        pltpu.VMEM((16,), jnp.float32),
        pltpu.VMEM((NPAD,), jnp.float32),
        pltpu.VMEM((2048,), jnp.float32),
        pltpu.VMEM((CH1,), jnp.int32),
        pltpu.VMEM((CH1 // 128, 128), jnp.int32),
        pltpu.VMEM((CH3,), jnp.int32),
        pltpu.VMEM((CH3 // 128, 128), jnp.int32),
        pltpu.VMEM((CH3,), jnp.float32),
        pltpu.VMEM((CH3, FW), jnp.float32),
        pltpu.VMEM((CH3,), jnp.int32),
        pltpu.VMEM((CH3 // 128, 128), jnp.int32),
        pltpu.VMEM((CH3,), jnp.float32),
        pltpu.VMEM((CH3, FW), jnp.float32),
        pltpu.VMEM((16, FW), jnp.float32),
        pltpu.VMEM_SHARED((NTILES, NPAD), jnp.float32),
        pltpu.VMEM_SHARED((NPAD, FW), jnp.float32),
        pltpu.SemaphoreType.DMA,
        pltpu.SemaphoreType.DMA,
        pltpu.SemaphoreType.DMA,
        pltpu.SemaphoreType.DMA,
    ),
)


def _mm_body(pre, split, p_ref, b_ref, w_ref, a_ref,
             hlo_ref, hhi_ref, hsd_ref):
    if pre:
        x = p_ref[0]
    else:
        x = _lrelu(p_ref[...] + b_ref[0])
        rid = lax.broadcasted_iota(jnp.int32, (NPAD, 1), 0)
        x = jnp.where(rid < N, x, 0.0)
    h = x @ w_ref[0]
    hsd = h @ a_ref[0]
    mxv = jnp.max(hsd[:, 0])
    ci = lax.broadcasted_iota(jnp.int32, (NPAD, 4), 1)
    hsd = jnp.where(ci == 2, mxv, hsd)
    if split:
        hlo_ref[...] = h[:, :FW]
        hhi_ref[...] = h[:, FW:]
    else:
        hlo_ref[...] = h
        hhi_ref[...] = jnp.zeros((NPAD, FW), jnp.float32)
    hsd_ref[...] = hsd


def _k_mm(pre, p, b, W, A):
    Fi, Fo = W.shape[1], W.shape[2]
    split = Fo == 2 * FW
    p_spec = (pl.BlockSpec((1, NPAD, Fi), lambda i: (i, 0, 0)) if pre
              else pl.BlockSpec((NPAD, Fi), lambda i: (i, 0)))
    return pl.pallas_call(
        functools.partial(_mm_body, pre, split),
        grid=(2,),
        in_specs=[p_spec,
                  pl.BlockSpec((1, 1, Fi), lambda i: (i, 0, 0)),
                  pl.BlockSpec((1, Fi, Fo), lambda i: (i, 0, 0)),
                  pl.BlockSpec((1, Fo, 4), lambda i: (i, 0, 0))],
        out_specs=[pl.BlockSpec((NPAD, FW), lambda i: (i, 0)),
                   pl.BlockSpec((NPAD, FW), lambda i: (i, 0)),
                   pl.BlockSpec((NPAD, 4), lambda i: (i, 0))],
        out_shape=[jax.ShapeDtypeStruct((2 * NPAD, FW), jnp.float32),
                   jax.ShapeDtypeStruct((2 * NPAD, FW), jnp.float32),
                   jax.ShapeDtypeStruct((2 * NPAD, 4), jnp.float32)],
    )(p, b, W, A)


def _post_body(plo_ref, phi_ref, b_ref, aw_ref, ab_ref, bc_ref,
               f1_ref, fb1_ref, g1_ref, be1_ref, f2_ref, fb2_ref, hf_ref):
    p = jnp.concatenate([plo_ref[...], phi_ref[...]], axis=1)
    x = _lrelu(p + b_ref[0])
    rid = lax.broadcasted_iota(jnp.int32, (NPAD, 1), 0)
    valid = rid < N
    x = jnp.where(valid, x, 0.0)
    l = x @ aw_ref[0] + ab_ref[0]
    lm = jnp.max(jnp.where(valid, l, -1e30))
    u = jnp.where(valid, jnp.exp(l - lm), 0.0)
    den = jnp.sum(u)
    onehot = (bc_ref[0] == lax.broadcasted_iota(jnp.int32, (1, NG), 1))
    w = jnp.where(onehot, u, 0.0)
    g = lax.dot_general(w, x, (((0,), (0,)), ((), ()))) / den
    hh = _lrelu((g @ f1_ref[0] + fb1_ref[0]) * g1_ref[0] + be1_ref[0])
    hf_ref[0] = hh @ f2_ref[0] + fb2_ref[0]


def _k_post(plo, phi, b, aw, ab, bc, f1, fb1, g1, be1, f2, fb2):
    args = (b, aw, ab, bc, f1, fb1, g1, be1, f2, fb2)
    specs = [pl.BlockSpec((NPAD, FW), lambda i: (i, 0)),
             pl.BlockSpec((NPAD, FW), lambda i: (i, 0))]
    specs += [pl.BlockSpec((1,) + x.shape[1:],
                           lambda i, n=x.ndim: (i,) + (0,) * (n - 1))
              for x in args]
    return pl.pallas_call(
        _post_body,
        grid=(2,),
        in_specs=specs,
        out_specs=pl.BlockSpec((1, NG, 128), lambda i: (i, 0, 0)),
        out_shape=jax.ShapeDtypeStruct((2, NG, 128), jnp.float32),
    )(plo, phi, *args)


def _tail_body(hf_ref, cell_ref, cp, mp, sp, out_ref):
    cw1, cb1, cg1, cbe1, cw2, cb2, cg2, cbe2, cw3, cb3 = cp
    h1 = hf_ref[0]
    h2 = hf_ref[1]
    cell = cell_ref[...]
    v = cell / (jnp.sqrt(jnp.sum(cell * cell, axis=1, keepdims=True)) + 1e-12)
    c = _lrelu((v @ cw1[...] + cb1[...]) * cg1[...] + cbe1[...])
    c = _lrelu((c @ cw2[...] + cb2[...]) * cg2[...] + cbe2[...])
    c = c @ cw3[...] + cb3[...]
    x = jnp.concatenate([h1, h2, c], axis=1)
    for i in range(2):
        gw, gb, nw, nb, lw, lb = mp[6 * i:6 * i + 6]
        gate = jax.nn.sigmoid(x @ gw[...] + gb[...])
        nl = _lrelu(x @ nw[...] + nb[...])
        lin = x @ lw[...] + lb[...]
        x = gate * nl + (1.0 - gate) * lin
    w1, b1, w2, b2, w3, b3 = sp
    h = _lrelu(x @ w1[...] + b1[...])
    h = _lrelu(h @ w2[...] + b2[...])
    out_ref[...] = h @ w3[...] + b3[...]


def _k_tail(hf, cell, params):
    cp = params["cell"]
    mp = params["mfic"]
    sp = params["syn"]
    cp_list = [cp["w1"], cp["b1"], cp["g1"], cp["be1"], cp["w2"], cp["b2"],
               cp["g2"], cp["be2"], cp["w3"], cp["b3"]]
    mp_list = []
    for i in range(2):
        mp_list += [mp[f"gw{i}"], mp[f"gb{i}"], mp[f"nw{i}"], mp[f"nb{i}"],
                    mp[f"lw{i}"], mp[f"lb{i}"]]
    sp_list = [sp["w1"], sp["b1"], sp["w2"], sp["b2"], sp["w3"], sp["b3"]]
    return pl.pallas_call(
        _tail_body,
        out_shape=jax.ShapeDtypeStruct((NG, 2), jnp.float32),
    )(hf, cell, cp_list, mp_list, sp_list)


def _edge_arrays(ei):
    loop = jnp.arange(N, dtype=jnp.int32)
    pad = jnp.full((E_PAD - E_REAL,), NPAD - 1, jnp.int32)
    src = jnp.concatenate([ei[0], loop, pad])
    dst = jnp.concatenate([ei[1], loop, pad])
    return src, dst


def kernel(x1, edge_index1, batch1, x2, edge_index2, batch2, cell, params):
    d1, d2 = params["d1"], params["d2"]
    src1, dst1 = _edge_arrays(edge_index1)
    src2, dst2 = _edge_arrays(edge_index2)
    src = jnp.concatenate([src1, src2])
    dst2d = jnp.concatenate([dst1, dst2]).reshape(-1, 128)
    xp = jnp.zeros((2, NPAD, 78), jnp.float32)
    xp = xp.at[:, :N].set(jnp.stack([x1, x2]))
    bc = jnp.stack([batch1, batch2]).astype(jnp.int32)
    bc = jnp.pad(bc, ((0, 0), (0, NPAD - N)))[:, :, None]

    def st(name):
        return jnp.stack([d1[name], d2[name]])

    Ws = [st(f"W{i}") for i in range(3)]
    Ws[0] = jnp.pad(Ws[0], ((0, 0), (0, 0), (0, 32)))
    Ws[1] = jnp.pad(Ws[1], ((0, 0), (0, 32), (0, 0)))
    As = []
    for i in range(3):
        pads = jnp.zeros_like(d1[f"as{i}"])
        As.append(jnp.stack(
            [jnp.stack([d1[f"as{i}"], d1[f"ad{i}"], pads, pads], axis=1),
             jnp.stack([d2[f"as{i}"], d2[f"ad{i}"], pads, pads], axis=1)]))
    As[0] = jnp.pad(As[0], ((0, 0), (0, 32), (0, 0)))
    bs = [st(f"b{i}") for i in range(3)]
    bs[0] = jnp.pad(bs[0], ((0, 0), (0, 32)))

    hlo, hhi, hsd = _k_mm(True, xp, jnp.zeros((2, 1, 78), jnp.float32),
                          Ws[0], As[0])
    out0 = _conv(src, dst2d, hsd[:, 0], hsd[:, 1], hsd[:, 2], hlo)
    hlo, hhi, hsd = _k_mm(False, out0, bs[0][:, None], Ws[1], As[1])
    out1 = _conv(src, dst2d, hsd[:, 0], hsd[:, 1], hsd[:, 2], hlo)
    hlo, hhi, hsd = _k_mm(False, out1, bs[1][:, None], Ws[2], As[2])
    out_lo = _conv(src, dst2d, hsd[:, 0], hsd[:, 1], hsd[:, 2], hlo)
    out_hi = _conv(src, dst2d, hsd[:, 0], hsd[:, 1], hsd[:, 2], hhi)

    hf = _k_post(out_lo, out_hi, bs[2][:, None], st("att_w"),
                 st("att_b")[:, None], bc,
                 st("fc1_w"), st("fc1_b")[:, None], st("bn1_g")[:, None],
                 st("bn1_b")[:, None], st("fc2_w"), st("fc2_b")[:, None])

    return _k_tail(hf, cell, params)


# speedup vs baseline: 38.4760x; 1.6392x over previous
import functools

import jax
import jax.numpy as jnp
from jax import lax
from jax.experimental import pallas as pl
from jax.experimental.pallas import tpu as pltpu
from jax.experimental.pallas import tpu_sc as plsc

NG = 128
N = 10000
NPAD = 10240
E_REAL = 650000
NTILES = 16
EPT = 40960
E_PAD = EPT * NTILES
SL = NPAD // NTILES
CH1 = 2048
CH3 = 256
FW = 64

_MESH = plsc.VectorSubcoreMesh(
    core_axis_name="c", subcore_axis_name="s", num_cores=2, num_subcores=16)
_SC_PARAMS = pltpu.CompilerParams(
    use_tc_tiling_on_sc=False, needs_layout_passes=False)


def _lrelu(x, s=0.01):
    return jnp.where(x >= 0, x, s * x)


def _conv_body(src_hbm, dst2_hbm, hs_hbm, hd_hbm, mx_hbm, h_hbm,
               out_hbm,
               hs_v, hd_v, mx_v, den_v, dent_v, src1_v, dst1_v,
               src_a, dst2_a, al_a, rows_a,
               src_b, dst2_b, al_b, rows_b, zb_v,
               sh_den, sh_out, sga, sgb, ssa, ssb):
    c = lax.axis_index("c")
    s = lax.axis_index("s")
    iota16 = lax.iota(jnp.int32, 16)
    zeros16 = jnp.zeros((16,), jnp.float32)
    zeros16i = jnp.zeros((16,), jnp.int32)

    pltpu.sync_copy(hs_hbm.at[pl.ds(c * NPAD, NPAD)], hs_v)
    pltpu.sync_copy(hd_hbm.at[pl.ds(c * NPAD, NPAD)], hd_v)
    pltpu.sync_copy(mx_hbm.at[pl.ds(c * NPAD, 16)], mx_v)
    mx = mx_v[...]

    def zb0(i, carry):
        den_v[pl.ds(i * 16, 16)] = zeros16
        return carry
    lax.fori_loop(0, NPAD // 16, zb0, 0)

    ebase = (c * NTILES + s) * EPT

    def p1(ci, carry):
        off = ebase + ci * CH1
        pltpu.sync_copy(src_hbm.at[pl.ds(off, CH1)], src1_v)
        pltpu.sync_copy(dst2_hbm.at[pl.ds(off // 128, CH1 // 128)], dst1_v)

        def eb(g, carry2):
            isrc = src1_v[pl.ds(g * 16, 16)]
            idst = plsc.load_gather(
                dst1_v, [jnp.full((16,), g // 8, jnp.int32),
                         (g % 8) * 16 + iota16])
            a = plsc.load_gather(hs_v, [isrc])
            b = plsc.load_gather(hd_v, [idst])
            z = a + b
            e = jnp.where(z >= 0.0, z, 0.2 * z)
            m = jnp.maximum(b + mx, 0.0)
            plsc.addupdate_scatter(den_v, [idst], jnp.exp(e - m))
            return carry2
        lax.fori_loop(0, CH1 // 16, eb, 0)
        return carry
    lax.fori_loop(0, EPT // CH1, p1, 0)

    pltpu.sync_copy(den_v, sh_den.at[s])

    def zq(i, carry):
        plsc.store_scatter(zb_v, [zeros16i + i // (FW // 16),
                                  (i % (FW // 16)) * 16 + iota16], zeros16)
        return carry
    lax.fori_loop(0, 16 * FW // 16, zq, 0)
    for k in range(SL // 16):
        pltpu.sync_copy(zb_v, sh_out.at[pl.ds(s * SL + k * 16, 16)])
    plsc.subcore_barrier()

    def zb1(i, carry):
        den_v[pl.ds(i * 16, 16)] = zeros16
        return carry
    lax.fori_loop(0, NPAD // 16, zb1, 0)
    for t in range(NTILES):
        for q in range(NPAD // 2048):
            pltpu.sync_copy(sh_den.at[t, pl.ds(q * 2048, 2048)], dent_v)

            def ad(i, carry):
                sl = pl.ds(q * 2048 + i * 16, 16)
                sl2 = pl.ds(i * 16, 16)
                den_v[sl] = den_v[sl] + dent_v[sl2]
                return carry
            lax.fori_loop(0, 2048 // 16, ad, 0)

    coff = c * NPAD
    NCH = EPT // CH3

    def prep(i, srcb, dst2b, alb):
        off = ebase + i * CH3
        pltpu.sync_copy(src_hbm.at[pl.ds(off, CH3)], srcb)
        pltpu.sync_copy(dst2_hbm.at[pl.ds(off // 128, CH3 // 128)], dst2b)

        def ab(g, carry2):
            isrc = srcb[pl.ds(g * 16, 16)]
            idst = plsc.load_gather(
                dst2b, [jnp.full((16,), g // 8, jnp.int32),
                        (g % 8) * 16 + iota16])
            a = plsc.load_gather(hs_v, [isrc])
            b = plsc.load_gather(hd_v, [idst])
            z = a + b
            e = jnp.where(z >= 0.0, z, 0.2 * z)
            m = jnp.maximum(b + mx, 0.0)
            exv = jnp.exp(e - m)
            dv = plsc.load_gather(den_v, [idst])
            alb[pl.ds(g * 16, 16)] = exv / (dv + 1e-16)
            srcb[pl.ds(g * 16, 16)] = isrc + coff
            return carry2
        lax.fori_loop(0, CH3 // 16, ab, 0)

    def scale(rowsb, alb):
        def scg(g, carry2):
            alv = alb[pl.ds(g * 16, 16)]
            for k in range(16):
                e_row = g * 16 + k
                bc = lax.gather(
                    alv, jnp.full((16, 1), k, jnp.int32),
                    lax.GatherDimensionNumbers(
                        offset_dims=(), collapsed_slice_dims=(0,),
                        start_index_map=(0,)),
                    (1,), mode=lax.GatherScatterMode.PROMISE_IN_BOUNDS)
                for j in range(FW // 16):
                    sl2 = pl.ds(j * 16, 16)
                    rowsb[e_row, sl2] = rowsb[e_row, sl2] * bc
            return carry2
        lax.fori_loop(0, CH3 // 16, scg, 0)

    def fire_gather(srcb, rowsb, semx):
        pltpu.async_copy(h_hbm.at[srcb], rowsb, semx)

    def drain_gather(rowsb, semx):
        pltpu.make_async_copy(h_hbm.at[pl.ds(0, CH3)], rowsb, semx).wait()

    def fire_scatters(rowsb, dst2b, semx):
        for j in range(CH3 // 128):
            pltpu.async_copy(rowsb.at[pl.ds(j * 128, 128)],
                             sh_out.at[dst2b.at[j]], semx, add=True)

    def drain_scatters(rowsb, semx):
        for j in range(CH3 // 128):
            pltpu.make_async_copy(h_hbm.at[pl.ds(0, 128)],
                                  rowsb.at[pl.ds(j * 128, 128)], semx).wait()

    prep(0, src_a, dst2_a, al_a)
    fire_gather(src_a, rows_a, sga)
    prep(1, src_b, dst2_b, al_b)
    fire_gather(src_b, rows_b, sgb)

    def body(cj, carry):
        i0 = 2 * cj
        drain_gather(rows_a, sga)
        scale(rows_a, al_a)
        fire_scatters(rows_a, dst2_a, ssa)
        drain_gather(rows_b, sgb)
        scale(rows_b, al_b)
        fire_scatters(rows_b, dst2_b, ssb)
        drain_scatters(rows_a, ssa)
        prep(jnp.minimum(i0 + 2, NCH - 1), src_a, dst2_a, al_a)
        fire_gather(src_a, rows_a, sga)
        drain_scatters(rows_b, ssb)
        prep(jnp.minimum(i0 + 3, NCH - 1), src_b, dst2_b, al_b)
        fire_gather(src_b, rows_b, sgb)
        return carry
    lax.fori_loop(0, NCH // 2, body, 0)
    drain_gather(rows_a, sga)
    drain_gather(rows_b, sgb)

    plsc.subcore_barrier()
    pltpu.sync_copy(sh_out.at[pl.ds(s * SL, SL)],
                    out_hbm.at[pl.ds(c * NPAD + s * SL, SL)])


_conv = pl.kernel(
    _conv_body,
    out_type=jax.ShapeDtypeStruct((2 * NPAD, FW), jnp.float32),
    mesh=_MESH,
    compiler_params=_SC_PARAMS,
    scratch_types=(
        pltpu.VMEM((NPAD,), jnp.float32),
        pltpu.VMEM((NPAD,), jnp.float32),
        pltpu.VMEM((16,), jnp.float32),
        pltpu.VMEM((NPAD,), jnp.float32),
        pltpu.VMEM((2048,), jnp.float32),
        pltpu.VMEM((CH1,), jnp.int32),
        pltpu.VMEM((CH1 // 128, 128), jnp.int32),
        pltpu.VMEM((CH3,), jnp.int32),
        pltpu.VMEM((CH3 // 128, 128), jnp.int32),
        pltpu.VMEM((CH3,), jnp.float32),
        pltpu.VMEM((CH3, FW), jnp.float32),
        pltpu.VMEM((CH3,), jnp.int32),
        pltpu.VMEM((CH3 // 128, 128), jnp.int32),
        pltpu.VMEM((CH3,), jnp.float32),
        pltpu.VMEM((CH3, FW), jnp.float32),
        pltpu.VMEM((16, FW), jnp.float32),
        pltpu.VMEM_SHARED((NTILES, NPAD), jnp.float32),
        pltpu.VMEM_SHARED((NPAD, FW), jnp.float32),
        pltpu.SemaphoreType.DMA,
        pltpu.SemaphoreType.DMA,
        pltpu.SemaphoreType.DMA,
        pltpu.SemaphoreType.DMA,
    ),
)


def _mm_body(pre, split, p_ref, b_ref, w_ref, a_ref,
             hlo_ref, hhi_ref, hsd_ref):
    if pre:
        x = p_ref[0]
    else:
        x = _lrelu(p_ref[...] + b_ref[0])
        rid = lax.broadcasted_iota(jnp.int32, (NPAD, 1), 0)
        x = jnp.where(rid < N, x, 0.0)
    h = x @ w_ref[0]
    hsd = h @ a_ref[0]
    mxv = jnp.max(hsd[:, 0])
    ci = lax.broadcasted_iota(jnp.int32, (NPAD, 4), 1)
    hsd = jnp.where(ci == 2, mxv, hsd)
    if split:
        hlo_ref[...] = h[:, :FW]
        hhi_ref[...] = h[:, FW:]
    else:
        hlo_ref[...] = h
        hhi_ref[...] = jnp.zeros((NPAD, FW), jnp.float32)
    hsd_ref[...] = hsd


def _k_mm(pre, p, b, W, A):
    Fi, Fo = W.shape[1], W.shape[2]
    split = Fo == 2 * FW
    p_spec = (pl.BlockSpec((1, NPAD, Fi), lambda i: (i, 0, 0)) if pre
              else pl.BlockSpec((NPAD, Fi), lambda i: (i, 0)))
    return pl.pallas_call(
        functools.partial(_mm_body, pre, split),
        grid=(2,),
        in_specs=[p_spec,
                  pl.BlockSpec((1, 1, Fi), lambda i: (i, 0, 0)),
                  pl.BlockSpec((1, Fi, Fo), lambda i: (i, 0, 0)),
                  pl.BlockSpec((1, Fo, 4), lambda i: (i, 0, 0))],
        out_specs=[pl.BlockSpec((NPAD, FW), lambda i: (i, 0)),
                   pl.BlockSpec((NPAD, FW), lambda i: (i, 0)),
                   pl.BlockSpec((NPAD, 4), lambda i: (i, 0))],
        out_shape=[jax.ShapeDtypeStruct((2 * NPAD, FW), jnp.float32),
                   jax.ShapeDtypeStruct((2 * NPAD, FW), jnp.float32),
                   jax.ShapeDtypeStruct((2 * NPAD, 4), jnp.float32)],
    )(p, b, W, A)


def _post_body(plo_ref, phi_ref, b_ref, aw_ref, ab_ref, bc_ref,
               f1_ref, fb1_ref, g1_ref, be1_ref, f2_ref, fb2_ref, hf_ref):
    p = jnp.concatenate([plo_ref[...], phi_ref[...]], axis=1)
    x = _lrelu(p + b_ref[0])
    rid = lax.broadcasted_iota(jnp.int32, (NPAD, 1), 0)
    valid = rid < N
    x = jnp.where(valid, x, 0.0)
    l = x @ aw_ref[0] + ab_ref[0]
    lm = jnp.max(jnp.where(valid, l, -1e30))
    u = jnp.where(valid, jnp.exp(l - lm), 0.0)
    den = jnp.sum(u)
    onehot = (bc_ref[0] == lax.broadcasted_iota(jnp.int32, (1, NG), 1))
    w = jnp.where(onehot, u, 0.0)
    g = lax.dot_general(w, x, (((0,), (0,)), ((), ()))) / den
    hh = _lrelu((g @ f1_ref[0] + fb1_ref[0]) * g1_ref[0] + be1_ref[0])
    hf_ref[0] = hh @ f2_ref[0] + fb2_ref[0]


def _k_post(plo, phi, b, aw, ab, bc, f1, fb1, g1, be1, f2, fb2):
    args = (b, aw, ab, bc, f1, fb1, g1, be1, f2, fb2)
    specs = [pl.BlockSpec((NPAD, FW), lambda i: (i, 0)),
             pl.BlockSpec((NPAD, FW), lambda i: (i, 0))]
    specs += [pl.BlockSpec((1,) + x.shape[1:],
                           lambda i, n=x.ndim: (i,) + (0,) * (n - 1))
              for x in args]
    return pl.pallas_call(
        _post_body,
        grid=(2,),
        in_specs=specs,
        out_specs=pl.BlockSpec((1, NG, 128), lambda i: (i, 0, 0)),
        out_shape=jax.ShapeDtypeStruct((2, NG, 128), jnp.float32),
    )(plo, phi, *args)


def _tail_body(hf_ref, cell_ref, cp, mp, sp, out_ref):
    cw1, cb1, cg1, cbe1, cw2, cb2, cg2, cbe2, cw3, cb3 = cp
    h1 = hf_ref[0]
    h2 = hf_ref[1]
    cell = cell_ref[...]
    v = cell / (jnp.sqrt(jnp.sum(cell * cell, axis=1, keepdims=True)) + 1e-12)
    c = _lrelu((v @ cw1[...] + cb1[...]) * cg1[...] + cbe1[...])
    c = _lrelu((c @ cw2[...] + cb2[...]) * cg2[...] + cbe2[...])
    c = c @ cw3[...] + cb3[...]
    x = jnp.concatenate([h1, h2, c], axis=1)
    for i in range(2):
        gw, gb, nw, nb, lw, lb = mp[6 * i:6 * i + 6]
        gate = jax.nn.sigmoid(x @ gw[...] + gb[...])
        nl = _lrelu(x @ nw[...] + nb[...])
        lin = x @ lw[...] + lb[...]
        x = gate * nl + (1.0 - gate) * lin
    w1, b1, w2, b2, w3, b3 = sp
    h = _lrelu(x @ w1[...] + b1[...])
    h = _lrelu(h @ w2[...] + b2[...])
    out_ref[...] = h @ w3[...] + b3[...]


def _k_tail(hf, cell, params):
    cp = params["cell"]
    mp = params["mfic"]
    sp = params["syn"]
    cp_list = [cp["w1"], cp["b1"], cp["g1"], cp["be1"], cp["w2"], cp["b2"],
               cp["g2"], cp["be2"], cp["w3"], cp["b3"]]
    mp_list = []
    for i in range(2):
        mp_list += [mp[f"gw{i}"], mp[f"gb{i}"], mp[f"nw{i}"], mp[f"nb{i}"],
                    mp[f"lw{i}"], mp[f"lb{i}"]]
    sp_list = [sp["w1"], sp["b1"], sp["w2"], sp["b2"], sp["w3"], sp["b3"]]
    return pl.pallas_call(
        _tail_body,
        out_shape=jax.ShapeDtypeStruct((NG, 2), jnp.float32),
    )(hf, cell, cp_list, mp_list, sp_list)


def _edge_arrays(ei):
    loop = jnp.arange(N, dtype=jnp.int32)
    pad = jnp.full((E_PAD - E_REAL,), NPAD - 1, jnp.int32)
    src = jnp.concatenate([ei[0], loop, pad])
    dst = jnp.concatenate([ei[1], loop, pad])
    return src, dst


def kernel(x1, edge_index1, batch1, x2, edge_index2, batch2, cell, params):
    d1, d2 = params["d1"], params["d2"]
    src1, dst1 = _edge_arrays(edge_index1)
    src2, dst2 = _edge_arrays(edge_index2)
    src = jnp.concatenate([src1, src2])
    dst2d = jnp.concatenate([dst1, dst2]).reshape(-1, 128)
    xp = jnp.zeros((2, NPAD, 78), jnp.float32)
    xp = xp.at[:, :N].set(jnp.stack([x1, x2]))
    bc = jnp.stack([batch1, batch2]).astype(jnp.int32)
    bc = jnp.pad(bc, ((0, 0), (0, NPAD - N)))[:, :, None]

    def st(name):
        return jnp.stack([d1[name], d2[name]])

    Ws = [st(f"W{i}") for i in range(3)]
    Ws[0] = jnp.pad(Ws[0], ((0, 0), (0, 0), (0, 32)))
    Ws[1] = jnp.pad(Ws[1], ((0, 0), (0, 32), (0, 0)))
    As = []
    for i in range(3):
        pads = jnp.zeros_like(d1[f"as{i}"])
        As.append(jnp.stack(
            [jnp.stack([d1[f"as{i}"], d1[f"ad{i}"], pads, pads], axis=1),
             jnp.stack([d2[f"as{i}"], d2[f"ad{i}"], pads, pads], axis=1)]))
    As[0] = jnp.pad(As[0], ((0, 0), (0, 32), (0, 0)))
    bs = [st(f"b{i}") for i in range(3)]
    bs[0] = jnp.pad(bs[0], ((0, 0), (0, 32)))

    hlo, hhi, hsd = _k_mm(True, xp, jnp.zeros((2, 1, 78), jnp.float32),
                          Ws[0], As[0])
    out0 = _conv(src, dst2d, hsd[:, 0], hsd[:, 1], hsd[:, 2], hlo)
    hlo, hhi, hsd = _k_mm(False, out0, bs[0][:, None], Ws[1], As[1])
    out1 = _conv(src, dst2d, hsd[:, 0], hsd[:, 1], hsd[:, 2], hlo)
    hlo, hhi, hsd = _k_mm(False, out1, bs[1][:, None], Ws[2], As[2])
    out_lo = _conv(src, dst2d, hsd[:, 0], hsd[:, 1], hsd[:, 2], hlo)
    out_hi = _conv(src, dst2d, hsd[:, 0], hsd[:, 1], hsd[:, 2], hhi)

    hf = _k_post(out_lo, out_hi, bs[2][:, None], st("att_w"),
                 st("att_b")[:, None], bc,
                 st("fc1_w"), st("fc1_b")[:, None], st("bn1_g")[:, None],
                 st("bn1_b")[:, None], st("fc2_w"), st("fc2_b")[:, None])

    return _k_tail(hf, cell, params)
